# baseline (speedup 1.0000x reference)
import functools

import jax
import jax.numpy as jnp
from jax.experimental import pallas as pl

BEAT_LEN = 128
TOKEN_DIM = 64


def _proj_kernel(x_ref, wt_ref, b_ref, o_ref):
    o_ref[...] = (
        jnp.dot(x_ref[...], wt_ref[...], preferred_element_type=jnp.float32)
        + b_ref[...]
    )


@functools.partial(jax.jit, static_argnames=())
def _run(ecg, W, b):
    B, L, T = ecg.shape
    N = T // BEAT_LEN
    M = B * L * N
    x = ecg.reshape(M, BEAT_LEN)
    wt = W.T
    b2 = b.reshape(1, TOKEN_DIM)

    BLK_M = 768
    grid = (M // BLK_M,)
    out = pl.pallas_call(
        _proj_kernel,
        grid=grid,
        in_specs=[
            pl.BlockSpec((BLK_M, BEAT_LEN), lambda i: (i, 0)),
            pl.BlockSpec((BEAT_LEN, TOKEN_DIM), lambda i: (0, 0)),
            pl.BlockSpec((1, TOKEN_DIM), lambda i: (0, 0)),
        ],
        out_specs=pl.BlockSpec((BLK_M, TOKEN_DIM), lambda i: (i, 0)),
        out_shape=jax.ShapeDtypeStruct((M, TOKEN_DIM), jnp.float32),
    )(x, wt, b2)

    X = out.reshape(B, L, N, TOKEN_DIM)
    beat_intervals = jnp.full((B, N), float(BEAT_LEN), dtype=jnp.float32)
    return (X, beat_intervals)


def kernel(ecg, W, b):
    return _run(ecg, W, b)


# speedup vs baseline: 1.2829x; 1.2829x over previous
import functools

import jax
import jax.numpy as jnp
from jax.experimental import pallas as pl

BEAT_LEN = 128
TOKEN_DIM = 64


def _proj_kernel(x_ref, wt_ref, b_ref, o_ref, bi_ref):
    o_ref[...] = (
        jnp.dot(x_ref[...], wt_ref[...], preferred_element_type=jnp.float32)
        + b_ref[...]
    )
    bi_ref[...] = jnp.full(bi_ref.shape, float(BEAT_LEN), dtype=jnp.float32)


@jax.jit
def _run(ecg, W, b):
    B, L, T = ecg.shape
    N = T // BEAT_LEN
    M = B * L * N
    x = ecg.reshape(M, BEAT_LEN)
    wt = W.T
    b2 = b.reshape(1, TOKEN_DIM)

    out, bi = pl.pallas_call(
        _proj_kernel,
        in_specs=[
            pl.BlockSpec((M, BEAT_LEN), lambda: (0, 0)),
            pl.BlockSpec((BEAT_LEN, TOKEN_DIM), lambda: (0, 0)),
            pl.BlockSpec((1, TOKEN_DIM), lambda: (0, 0)),
        ],
        out_specs=[
            pl.BlockSpec((M, TOKEN_DIM), lambda: (0, 0)),
            pl.BlockSpec((B, N), lambda: (0, 0)),
        ],
        out_shape=[
            jax.ShapeDtypeStruct((M, TOKEN_DIM), jnp.float32),
            jax.ShapeDtypeStruct((B, N), jnp.float32),
        ],
    )(x, wt, b2)

    X = out.reshape(B, L, N, TOKEN_DIM)
    return (X, bi)


def kernel(ecg, W, b):
    return _run(ecg, W, b)
